# baseline (speedup 1.0000x reference)
import functools

import jax
import jax.numpy as jnp
from jax import lax
from jax.experimental import pallas as pl
from jax.experimental.pallas import tpu as pltpu
from jax.experimental.pallas import tpu_sc as plsc

B = 16384
D = 64


def _sc_gather(P_table, Q_table, uidx, pidx):
    info = plsc.get_sparse_core_info()
    NC, NS, L = info.num_cores, info.num_subcores, info.num_lanes
    NW = NC * NS
    bpw = B // NW
    mesh = plsc.VectorSubcoreMesh(core_axis_name="c", subcore_axis_name="s")

    u2 = uidx.reshape(NW, bpw)
    p2 = pidx.reshape(NW, bpw)

    @functools.partial(
        pl.kernel,
        mesh=mesh,
        out_type=[
            jax.ShapeDtypeStruct((B, D), jnp.float32),
            jax.ShapeDtypeStruct((B, D), jnp.float32),
        ],
        scratch_types=[
            pltpu.VMEM((bpw,), jnp.int32),
            pltpu.VMEM((bpw,), jnp.int32),
            pltpu.SemaphoreType.DMA,
        ],
    )
    def k(P_hbm, Q_hbm, u_hbm, pr_hbm, Pout, Qout, uv, pv, sem):
        wid = lax.axis_index("s") * NC + lax.axis_index("c")
        base = wid * bpw
        pltpu.sync_copy(u_hbm.at[wid], uv)
        pltpu.sync_copy(pr_hbm.at[wid], pv)

        def body(i, _):
            uvec = uv[pl.ds(i * L, L)]
            pvec = pv[pl.ds(i * L, L)]
            copies = []
            for l in range(L):
                row = base + i * L + l
                copies.append(
                    pltpu.async_copy(
                        P_hbm.at[pl.ds(uvec[l], 1)], Pout.at[pl.ds(row, 1)], sem
                    )
                )
                copies.append(
                    pltpu.async_copy(
                        Q_hbm.at[pl.ds(pvec[l], 1)], Qout.at[pl.ds(row, 1)], sem
                    )
                )
            for c in copies:
                c.wait()
            return 0

        lax.fori_loop(0, bpw // L, body, 0)

    return k(P_table, Q_table, u2, p2)


def _mlp_body(p, q, w1a, w1b, b1, w2, b2, o):
    h = jnp.dot(p[...], w1a[...], preferred_element_type=jnp.float32)
    h = h + jnp.dot(q[...], w1b[...], preferred_element_type=jnp.float32)
    h = jnp.maximum(h + b1[...], 0.0)
    o[...] = jnp.sum(h * w2[...], axis=1, keepdims=True) + b2[...]


def _tc_mlp(P, Q, W1a, W1b, b1r, w2r, b2r):
    TB = 2048
    return pl.pallas_call(
        _mlp_body,
        grid=(B // TB,),
        in_specs=[
            pl.BlockSpec((TB, D), lambda i: (i, 0)),
            pl.BlockSpec((TB, D), lambda i: (i, 0)),
            pl.BlockSpec((D, D), lambda i: (0, 0)),
            pl.BlockSpec((D, D), lambda i: (0, 0)),
            pl.BlockSpec((1, D), lambda i: (0, 0)),
            pl.BlockSpec((1, D), lambda i: (0, 0)),
            pl.BlockSpec((1, 1), lambda i: (0, 0)),
        ],
        out_specs=pl.BlockSpec((TB, 1), lambda i: (i, 0)),
        out_shape=jax.ShapeDtypeStruct((B, 1), jnp.float32),
    )(P, Q, W1a, W1b, b1r, w2r, b2r)


def kernel(user, product, P_table, Q_table, W1, b1, W2, b2):
    user = user.astype(jnp.int32)
    product = product.astype(jnp.int32)
    P, Q = _sc_gather(P_table, Q_table, user, product)
    W1a = W1[:D]
    W1b = W1[D:]
    return _tc_mlp(
        P,
        Q,
        W1a,
        W1b,
        b1.reshape(1, D),
        W2.reshape(1, D),
        b2.reshape(1, 1),
    )
